# Initial kernel scaffold; baseline (speedup 1.0000x reference)
import jax
import jax.numpy as jnp
from jax.experimental import pallas as pl


def kernel(feat, adj, W1, W2, gamma, beta):
    raise NotImplementedError("write your pallas kernel here")


# speedup vs baseline: 1.0311x; 1.0311x over previous
import jax
import jax.numpy as jnp
from jax.experimental import pallas as pl
from jax.experimental.pallas import tpu as pltpu

_N = 10000
_TILE = 400


def _pass1_kernel(adj_ref, feat_ref, w1_ref, h_ref, sum_ref, sq_ref, x1_ref):
    @pl.when(pl.program_id(0) == 0)
    def _init():
        f = feat_ref[...].astype(jnp.bfloat16)
        w = w1_ref[...].astype(jnp.bfloat16)
        x1 = jnp.dot(f, w, preferred_element_type=jnp.float32)
        x1_ref[...] = x1.astype(jnp.bfloat16)
        sum_ref[...] = jnp.zeros_like(sum_ref)
        sq_ref[...] = jnp.zeros_like(sq_ref)

    a = adj_ref[...].astype(jnp.bfloat16)
    h = jnp.dot(a, x1_ref[...], preferred_element_type=jnp.float32)
    h = jnp.maximum(h, 0.0)
    h_ref[...] = h
    sum_ref[...] += jnp.sum(h, axis=0, keepdims=True)
    sq_ref[...] += jnp.sum(h * h, axis=0, keepdims=True)


def _pass2_kernel(adj_ref, h_ref, mean_ref, s_ref, beta_ref, w2_ref, out_ref, z_ref):
    @pl.when(pl.program_id(0) == 0)
    def _make_z():
        xb = (h_ref[...] - mean_ref[...]) * s_ref[...] + beta_ref[...]
        z = jnp.dot(xb.astype(jnp.bfloat16), w2_ref[...].astype(jnp.bfloat16),
                    preferred_element_type=jnp.float32)
        z_ref[...] = z.astype(jnp.bfloat16)

    a = adj_ref[...].astype(jnp.bfloat16)
    out_ref[...] = jnp.dot(a, z_ref[...], preferred_element_type=jnp.float32)


def kernel(feat, adj, W1, W2, gamma, beta):
    n, in_feat = feat.shape
    hid = W1.shape[1]
    out_feat = W2.shape[1]
    grid = n // _TILE

    h, col_sum, col_sq = pl.pallas_call(
        _pass1_kernel,
        grid=(grid,),
        in_specs=[
            pl.BlockSpec((_TILE, n), lambda i: (i, 0)),
            pl.BlockSpec((n, in_feat), lambda i: (0, 0)),
            pl.BlockSpec((in_feat, hid), lambda i: (0, 0)),
        ],
        out_specs=[
            pl.BlockSpec((_TILE, hid), lambda i: (i, 0)),
            pl.BlockSpec((1, hid), lambda i: (0, 0)),
            pl.BlockSpec((1, hid), lambda i: (0, 0)),
        ],
        out_shape=[
            jax.ShapeDtypeStruct((n, hid), jnp.float32),
            jax.ShapeDtypeStruct((1, hid), jnp.float32),
            jax.ShapeDtypeStruct((1, hid), jnp.float32),
        ],
        scratch_shapes=[pltpu.VMEM((n, hid), jnp.bfloat16)],
        compiler_params=pltpu.CompilerParams(
            dimension_semantics=("arbitrary",),
        ),
    )(adj, feat, W1)

    mean = (col_sum[0] / n)[None, :]
    var = col_sq[0] / n - mean[0] * mean[0]
    s = (gamma * jax.lax.rsqrt(var + 1e-5))[None, :]

    out = pl.pallas_call(
        _pass2_kernel,
        grid=(grid,),
        in_specs=[
            pl.BlockSpec((_TILE, n), lambda i: (i, 0)),
            pl.BlockSpec((n, hid), lambda i: (0, 0)),
            pl.BlockSpec((1, hid), lambda i: (0, 0)),
            pl.BlockSpec((1, hid), lambda i: (0, 0)),
            pl.BlockSpec((1, hid), lambda i: (0, 0)),
            pl.BlockSpec((hid, out_feat), lambda i: (0, 0)),
        ],
        out_specs=pl.BlockSpec((_TILE, out_feat), lambda i: (i, 0)),
        out_shape=jax.ShapeDtypeStruct((n, out_feat), jnp.float32),
        scratch_shapes=[pltpu.VMEM((n, out_feat), jnp.bfloat16)],
        compiler_params=pltpu.CompilerParams(
            dimension_semantics=("arbitrary",),
        ),
    )(adj, h, mean, s, beta[None, :], W2)

    return out


# speedup vs baseline: 1.0516x; 1.0198x over previous
import jax
import jax.numpy as jnp
from jax.experimental import pallas as pl
from jax.experimental.pallas import tpu as pltpu

_N = 10000
_TILE = 400
_EPS = 1e-5


def _fused_kernel(adj_ref, feat_ref, w1_ref, w2_ref, g_ref, b_ref, out_ref,
                  h_ref, x1_ref, z_ref, cs_ref, cq_ref):
    p = pl.program_id(0)
    i = pl.program_id(1)
    n = h_ref.shape[0]

    @pl.when((p == 0) & (i == 0))
    def _init():
        x1 = jnp.dot(feat_ref[...], w1_ref[...],
                     preferred_element_type=jnp.float32)
        x1_ref[...] = x1.astype(jnp.bfloat16)
        cs_ref[...] = jnp.zeros_like(cs_ref)
        cq_ref[...] = jnp.zeros_like(cq_ref)

    @pl.when(p == 0)
    def _phase0():
        a = adj_ref[...].astype(jnp.bfloat16)
        h = jnp.dot(a, x1_ref[...], preferred_element_type=jnp.float32)
        h = jnp.maximum(h, 0.0)
        h_ref[pl.ds(i * _TILE, _TILE), :] = h
        cs_ref[...] += jnp.sum(h, axis=0, keepdims=True)
        cq_ref[...] += jnp.sum(h * h, axis=0, keepdims=True)

    @pl.when((p == 1) & (i == 0))
    def _make_z():
        mean = cs_ref[...] / n
        var = cq_ref[...] / n - mean * mean
        s = g_ref[...] * jax.lax.rsqrt(var + _EPS)
        xb = (h_ref[...] - mean) * s + b_ref[...]
        z = jnp.dot(xb.astype(jnp.bfloat16), w2_ref[...],
                    preferred_element_type=jnp.float32)
        z_ref[...] = z.astype(jnp.bfloat16)

    @pl.when(p == 1)
    def _phase1():
        a = adj_ref[...].astype(jnp.bfloat16)
        out_ref[...] = jnp.dot(a, z_ref[...],
                               preferred_element_type=jnp.float32)


def kernel(feat, adj, W1, W2, gamma, beta):
    n, in_feat = feat.shape
    hid = W1.shape[1]
    out_feat = W2.shape[1]
    grid = n // _TILE

    out = pl.pallas_call(
        _fused_kernel,
        grid=(2, grid),
        in_specs=[
            pl.BlockSpec((_TILE, n), lambda p, i: (i, 0)),
            pl.BlockSpec((n, in_feat), lambda p, i: (0, 0)),
            pl.BlockSpec((in_feat, hid), lambda p, i: (0, 0)),
            pl.BlockSpec((hid, out_feat), lambda p, i: (0, 0)),
            pl.BlockSpec((1, hid), lambda p, i: (0, 0)),
            pl.BlockSpec((1, hid), lambda p, i: (0, 0)),
        ],
        out_specs=pl.BlockSpec((_TILE, out_feat), lambda p, i: (i, 0)),
        out_shape=jax.ShapeDtypeStruct((n, out_feat), jnp.float32),
        scratch_shapes=[
            pltpu.VMEM((n, hid), jnp.float32),
            pltpu.VMEM((n, hid), jnp.bfloat16),
            pltpu.VMEM((n, out_feat), jnp.bfloat16),
            pltpu.VMEM((1, hid), jnp.float32),
            pltpu.VMEM((1, hid), jnp.float32),
        ],
        compiler_params=pltpu.CompilerParams(
            dimension_semantics=("arbitrary", "arbitrary"),
        ),
    )(adj, feat.astype(jnp.bfloat16), W1.astype(jnp.bfloat16),
      W2.astype(jnp.bfloat16), gamma[None, :], beta[None, :])

    return out


# speedup vs baseline: 1.0523x; 1.0007x over previous
import jax
import jax.numpy as jnp
from jax.experimental import pallas as pl
from jax.experimental.pallas import tpu as pltpu

_N = 10000
_TILE = 400
_EPS = 1e-5


def _fused_kernel(adj_ref, feat_ref, w1_ref, w2_ref, g_ref, b_ref, out_ref,
                  h_ref, x1_ref, z_ref, cs_ref, cq_ref):
    p = pl.program_id(0)
    i = pl.program_id(1)
    n = h_ref.shape[0]

    @pl.when((p == 0) & (i == 0))
    def _init():
        x1 = jnp.dot(feat_ref[...], w1_ref[...],
                     preferred_element_type=jnp.float32)
        x1_ref[...] = x1.astype(jnp.bfloat16)
        cs_ref[...] = jnp.zeros_like(cs_ref)
        cq_ref[...] = jnp.zeros_like(cq_ref)

    @pl.when(p == 0)
    def _phase0():
        h = jax.lax.dot_general(
            adj_ref[...], x1_ref[...], (((1,), (0,)), ((), ())),
            precision=jax.lax.Precision.DEFAULT,
            preferred_element_type=jnp.float32)
        h = jnp.maximum(h, 0.0)
        h_ref[pl.ds(i * _TILE, _TILE), :] = h
        cs_ref[...] += jnp.sum(h, axis=0, keepdims=True)
        cq_ref[...] += jnp.sum(h * h, axis=0, keepdims=True)

    @pl.when((p == 1) & (i == 0))
    def _make_z():
        mean = cs_ref[...] / n
        var = cq_ref[...] / n - mean * mean
        s = g_ref[...] * jax.lax.rsqrt(var + _EPS)
        xb = (h_ref[...] - mean) * s + b_ref[...]
        z = jnp.dot(xb.astype(jnp.bfloat16), w2_ref[...],
                    preferred_element_type=jnp.float32)
        z_ref[...] = z.astype(jnp.bfloat16)

    @pl.when(p == 1)
    def _phase1():
        out_ref[...] = jax.lax.dot_general(
            adj_ref[...], z_ref[...], (((1,), (0,)), ((), ())),
            precision=jax.lax.Precision.DEFAULT,
            preferred_element_type=jnp.float32)


def kernel(feat, adj, W1, W2, gamma, beta):
    n, in_feat = feat.shape
    hid = W1.shape[1]
    out_feat = W2.shape[1]
    grid = n // _TILE

    out = pl.pallas_call(
        _fused_kernel,
        grid=(2, grid),
        in_specs=[
            pl.BlockSpec((_TILE, n), lambda p, i: (i, 0)),
            pl.BlockSpec((n, in_feat), lambda p, i: (0, 0)),
            pl.BlockSpec((in_feat, hid), lambda p, i: (0, 0)),
            pl.BlockSpec((hid, out_feat), lambda p, i: (0, 0)),
            pl.BlockSpec((1, hid), lambda p, i: (0, 0)),
            pl.BlockSpec((1, hid), lambda p, i: (0, 0)),
        ],
        out_specs=pl.BlockSpec((_TILE, out_feat), lambda p, i: (i, 0)),
        out_shape=jax.ShapeDtypeStruct((n, out_feat), jnp.float32),
        scratch_shapes=[
            pltpu.VMEM((n, hid), jnp.float32),
            pltpu.VMEM((n, hid), jnp.bfloat16),
            pltpu.VMEM((n, out_feat), jnp.bfloat16),
            pltpu.VMEM((1, hid), jnp.float32),
            pltpu.VMEM((1, hid), jnp.float32),
        ],
        compiler_params=pltpu.CompilerParams(
            dimension_semantics=("arbitrary", "arbitrary"),
        ),
    )(adj, feat.astype(jnp.bfloat16), W1.astype(jnp.bfloat16),
      W2.astype(jnp.bfloat16), gamma[None, :], beta[None, :])

    return out


# speedup vs baseline: 1.0530x; 1.0006x over previous
import jax
import jax.numpy as jnp
from jax.experimental import pallas as pl
from jax.experimental.pallas import tpu as pltpu

_N = 10000
_TILE = 400
_EPS = 1e-5


def _fused_kernel(adj_ref, adj2_ref, feat_ref, w1_ref, w2_ref, g_ref, b_ref, out_ref,
                  h_ref, x1_ref, z_ref, cs_ref, cq_ref):
    p = pl.program_id(0)
    i = pl.program_id(1)
    n = h_ref.shape[0]

    @pl.when((p == 0) & (i == 0))
    def _init():
        x1 = jnp.dot(feat_ref[...], w1_ref[...],
                     preferred_element_type=jnp.float32)
        x1_ref[...] = x1.astype(jnp.bfloat16)
        cs_ref[...] = jnp.zeros_like(cs_ref)
        cq_ref[...] = jnp.zeros_like(cq_ref)

    @pl.when(p == 0)
    def _phase0():
        half = _TILE // 2
        ha = jax.lax.dot_general(
            adj_ref[...], x1_ref[...], (((1,), (0,)), ((), ())),
            precision=jax.lax.Precision.DEFAULT,
            preferred_element_type=jnp.float32)
        hb = jax.lax.dot_general(
            adj2_ref[...], x1_ref[...], (((1,), (0,)), ((), ())),
            precision=jax.lax.Precision.DEFAULT,
            preferred_element_type=jnp.float32)
        ha = jnp.maximum(ha, 0.0)
        hb = jnp.maximum(hb, 0.0)
        h_ref[pl.ds(i * _TILE, half), :] = ha
        h_ref[pl.ds(i * _TILE + half, half), :] = hb
        cs_ref[...] += (jnp.sum(ha, axis=0, keepdims=True)
                        + jnp.sum(hb, axis=0, keepdims=True))
        cq_ref[...] += (jnp.sum(ha * ha, axis=0, keepdims=True)
                        + jnp.sum(hb * hb, axis=0, keepdims=True))

    @pl.when((p == 1) & (i == 0))
    def _make_z():
        mean = cs_ref[...] / n
        var = cq_ref[...] / n - mean * mean
        s = g_ref[...] * jax.lax.rsqrt(var + _EPS)
        xb = (h_ref[...] - mean) * s + b_ref[...]
        z = jnp.dot(xb.astype(jnp.bfloat16), w2_ref[...],
                    preferred_element_type=jnp.float32)
        z_ref[...] = z.astype(jnp.bfloat16)

    @pl.when(p == 1)
    def _phase1():
        half = _TILE // 2
        out_ref[pl.ds(0, half), :] = jax.lax.dot_general(
            adj_ref[...], z_ref[...], (((1,), (0,)), ((), ())),
            precision=jax.lax.Precision.DEFAULT,
            preferred_element_type=jnp.float32)
        out_ref[pl.ds(half, half), :] = jax.lax.dot_general(
            adj2_ref[...], z_ref[...], (((1,), (0,)), ((), ())),
            precision=jax.lax.Precision.DEFAULT,
            preferred_element_type=jnp.float32)


def kernel(feat, adj, W1, W2, gamma, beta):
    n, in_feat = feat.shape
    hid = W1.shape[1]
    out_feat = W2.shape[1]
    grid = n // _TILE

    out = pl.pallas_call(
        _fused_kernel,
        grid=(2, grid),
        in_specs=[
            pl.BlockSpec((_TILE // 2, n), lambda p, i: (2 * i, 0)),
            pl.BlockSpec((_TILE // 2, n), lambda p, i: (2 * i + 1, 0)),
            pl.BlockSpec((n, in_feat), lambda p, i: (0, 0)),
            pl.BlockSpec((in_feat, hid), lambda p, i: (0, 0)),
            pl.BlockSpec((hid, out_feat), lambda p, i: (0, 0)),
            pl.BlockSpec((1, hid), lambda p, i: (0, 0)),
            pl.BlockSpec((1, hid), lambda p, i: (0, 0)),
        ],
        out_specs=pl.BlockSpec((_TILE, out_feat), lambda p, i: (i, 0)),
        out_shape=jax.ShapeDtypeStruct((n, out_feat), jnp.float32),
        scratch_shapes=[
            pltpu.VMEM((n, hid), jnp.float32),
            pltpu.VMEM((n, hid), jnp.bfloat16),
            pltpu.VMEM((n, out_feat), jnp.bfloat16),
            pltpu.VMEM((1, hid), jnp.float32),
            pltpu.VMEM((1, hid), jnp.float32),
        ],
        compiler_params=pltpu.CompilerParams(
            dimension_semantics=("arbitrary", "arbitrary"),
        ),
    )(adj, adj, feat.astype(jnp.bfloat16), W1.astype(jnp.bfloat16),
      W2.astype(jnp.bfloat16), gamma[None, :], beta[None, :])

    return out


# speedup vs baseline: 1.0575x; 1.0043x over previous
import jax
import jax.numpy as jnp
from jax.experimental import pallas as pl
from jax.experimental.pallas import tpu as pltpu

_N = 10000
_TILE = 400
_EPS = 1e-5


def _fused_kernel(adj_ref, feat_ref, w1_ref, w2_ref, g_ref, b_ref, out_ref,
                  h_ref, x1_ref, z_ref, cs_ref, cq_ref):
    p = pl.program_id(0)
    i = pl.program_id(1)
    n = h_ref.shape[0]

    @pl.when((p == 0) & (i == 0))
    def _init():
        x1 = jnp.dot(feat_ref[...], w1_ref[...],
                     preferred_element_type=jnp.float32)
        x1_ref[...] = x1.astype(jnp.bfloat16)
        cs_ref[...] = jnp.zeros_like(cs_ref)
        cq_ref[...] = jnp.zeros_like(cq_ref)

    @pl.when(p == 0)
    def _phase0():
        h = jax.lax.dot_general(
            adj_ref[...], x1_ref[...], (((1,), (0,)), ((), ())),
            precision=jax.lax.Precision.DEFAULT,
            preferred_element_type=jnp.float32)
        h = jnp.maximum(h, 0.0)
        h_ref[pl.ds(i * _TILE, _TILE), :] = h
        cs_ref[...] += jnp.sum(h, axis=0, keepdims=True)
        cq_ref[...] += jnp.sum(h * h, axis=0, keepdims=True)

    @pl.when((p == 1) & (i == 0))
    def _make_z():
        mean = cs_ref[...] / n
        var = cq_ref[...] / n - mean * mean
        s = g_ref[...] * jax.lax.rsqrt(var + _EPS)
        xb = (h_ref[...] - mean) * s + b_ref[...]
        z = jnp.dot(xb.astype(jnp.bfloat16), w2_ref[...],
                    preferred_element_type=jnp.float32)
        z_ref[...] = z.astype(jnp.bfloat16)

    @pl.when(p == 1)
    def _phase1():
        out_ref[...] = jax.lax.dot_general(
            adj_ref[...], z_ref[...], (((1,), (0,)), ((), ())),
            precision=jax.lax.Precision.DEFAULT,
            preferred_element_type=jnp.float32)


def kernel(feat, adj, W1, W2, gamma, beta):
    n, in_feat = feat.shape
    hid = W1.shape[1]
    out_feat = W2.shape[1]
    grid = n // _TILE

    out = pl.pallas_call(
        _fused_kernel,
        grid=(2, grid),
        in_specs=[
            pl.BlockSpec((_TILE, n), lambda p, i: (i, 0)),
            pl.BlockSpec((n, in_feat), lambda p, i: (0, 0)),
            pl.BlockSpec((in_feat, hid), lambda p, i: (0, 0)),
            pl.BlockSpec((hid, out_feat), lambda p, i: (0, 0)),
            pl.BlockSpec((1, hid), lambda p, i: (0, 0)),
            pl.BlockSpec((1, hid), lambda p, i: (0, 0)),
        ],
        out_specs=pl.BlockSpec((_TILE, out_feat), lambda p, i: (p * i, 0)),
        out_shape=jax.ShapeDtypeStruct((n, out_feat), jnp.float32),
        scratch_shapes=[
            pltpu.VMEM((n, hid), jnp.float32),
            pltpu.VMEM((n, hid), jnp.bfloat16),
            pltpu.VMEM((n, out_feat), jnp.bfloat16),
            pltpu.VMEM((1, hid), jnp.float32),
            pltpu.VMEM((1, hid), jnp.float32),
        ],
        compiler_params=pltpu.CompilerParams(
            dimension_semantics=("arbitrary", "arbitrary"),
        ),
    )(adj, feat.astype(jnp.bfloat16), W1.astype(jnp.bfloat16),
      W2.astype(jnp.bfloat16), gamma[None, :], beta[None, :])

    return out
